# baseline (speedup 1.0000x reference)
import jax
import jax.numpy as jnp
from jax import lax
from jax.experimental import pallas as pl
from jax.experimental.pallas import tpu as pltpu
from jax.experimental.pallas import tpu_sc as plsc

L_OUT = 384
D = 256
ROWS_PER_WORKER = 12
WIN_ROWS = 408


def _pairwise_body(table_hbm, out_hbm, win, sem):
    c = lax.axis_index("c")
    s = lax.axis_index("s")
    wid = s * 2 + c
    a = wid * ROWS_PER_WORKER
    start = 489 - a
    ws = pl.multiple_of((start // 8) * 8, 8)
    off = start - ws
    pltpu.sync_copy(table_hbm.at[pl.ds(ws, WIN_ROWS)], win)
    copies = []
    for r in range(ROWS_PER_WORKER):
        copies.append(
            pltpu.async_copy(
                win.at[pl.ds(off + ROWS_PER_WORKER - 1 - r, L_OUT)],
                out_hbm.at[a + r],
                sem,
            )
        )
    for cp in copies:
        cp.wait()


def kernel(L, rel_pos_embed):
    mesh = plsc.VectorSubcoreMesh(core_axis_name="c", subcore_axis_name="s")
    run = pl.kernel(
        _pairwise_body,
        out_type=jax.ShapeDtypeStruct((L_OUT, L_OUT, D), jnp.float32),
        mesh=mesh,
        scratch_types=[
            pltpu.VMEM((WIN_ROWS, D), jnp.float32),
            pltpu.SemaphoreType.DMA,
        ],
        compiler_params=pltpu.CompilerParams(use_tc_tiling_on_sc=False),
    )
    return run(rel_pos_embed)


# speedup vs baseline: 2.8759x; 2.8759x over previous
import jax
import jax.numpy as jnp
from jax import lax
from jax.experimental import pallas as pl
from jax.experimental.pallas import tpu as pltpu
from jax.experimental.pallas import tpu_sc as plsc

L_OUT = 384
D = 256
ROWS_PER_WORKER = 12
T_ROWS = 992
WIN_ROWS = 472


def _pairwise_body(t_hbm, out_hbm, win, sem):
    c_ax = lax.axis_index("c")
    s_ax = lax.axis_index("s")
    wid = s_ax * 2 + c_ax
    c = wid % 8
    m = wid // 8
    p = (500 - c) % 8
    w0 = pl.multiple_of(412 - c - 96 * m - p, 8)
    pltpu.sync_copy(t_hbm.at[p, pl.ds(w0, WIN_ROWS)], win)
    copies = []
    for t in range(ROWS_PER_WORKER):
        copies.append(
            pltpu.async_copy(
                win.at[pl.ds(8 * (ROWS_PER_WORKER - 1 - t), L_OUT)],
                out_hbm.at[c + 96 * m + 8 * t],
                sem,
            )
        )
    for cp in copies:
        cp.wait()


def kernel(L, rel_pos_embed):
    t = jnp.stack([rel_pos_embed[p:p + T_ROWS] for p in range(8)], axis=0)
    mesh = plsc.VectorSubcoreMesh(core_axis_name="c", subcore_axis_name="s")
    run = pl.kernel(
        _pairwise_body,
        out_type=jax.ShapeDtypeStruct((L_OUT, L_OUT, D), jnp.float32),
        mesh=mesh,
        scratch_types=[
            pltpu.VMEM((WIN_ROWS, D), jnp.float32),
            pltpu.SemaphoreType.DMA,
        ],
    )
    return run(t)


# speedup vs baseline: 3.2099x; 1.1161x over previous
import jax
import jax.numpy as jnp
from jax import lax
from jax.experimental import pallas as pl
from jax.experimental.pallas import tpu as pltpu
from jax.experimental.pallas import tpu_sc as plsc

L_OUT = 384
D = 256
ROWS_PER_WORKER = 12
T_ROWS = 992
WIN_ROWS = 472


def _pairwise_body(t_hbm, out_hbm, win, sem):
    c_ax = lax.axis_index("c")
    s_ax = lax.axis_index("s")
    wid = s_ax * 2 + c_ax
    c = wid % 8
    m = wid // 8
    p = (500 - c) % 8
    w0 = pl.multiple_of(412 - c - 96 * m - p, 8)
    pltpu.sync_copy(t_hbm.at[p, pl.ds(w0, WIN_ROWS)], win)
    copies = []
    for t in range(ROWS_PER_WORKER):
        copies.append(
            pltpu.async_copy(
                win.at[pl.ds(8 * (ROWS_PER_WORKER - 1 - t), L_OUT)],
                out_hbm.at[c + 96 * m + 8 * t],
                sem,
            )
        )
    for cp in copies:
        cp.wait()


def _phase_body(table_ref, t_ref):
    for p in range(8):
        t_ref[p] = table_ref[pl.ds(p, T_ROWS), :]


def kernel(L, rel_pos_embed):
    t = pl.pallas_call(
        _phase_body,
        out_shape=jax.ShapeDtypeStruct((8, T_ROWS, D), jnp.float32),
    )(rel_pos_embed)
    mesh = plsc.VectorSubcoreMesh(core_axis_name="c", subcore_axis_name="s")
    run = pl.kernel(
        _pairwise_body,
        out_type=jax.ShapeDtypeStruct((L_OUT, L_OUT, D), jnp.float32),
        mesh=mesh,
        scratch_types=[
            pltpu.VMEM((WIN_ROWS, D), jnp.float32),
            pltpu.SemaphoreType.DMA,
        ],
    )
    return run(t)


# speedup vs baseline: 3.2172x; 1.0023x over previous
import jax
import jax.numpy as jnp
from jax import lax
from jax.experimental import pallas as pl
from jax.experimental.pallas import tpu as pltpu
from jax.experimental.pallas import tpu_sc as plsc

L_OUT = 384
D = 256
ROWS_PER_WORKER = 12
T_ROWS = 760
WIN_ROWS = 472


def _phase_body(table_ref, t_ref):
    for c in range(8):
        t_ref[c] = table_ref[pl.ds(124 - c, T_ROWS), :]


def _pairwise_body(t_hbm, out_hbm, win, sem):
    c_ax = lax.axis_index("c")
    s_ax = lax.axis_index("s")
    wid = s_ax * 2 + c_ax
    c = wid % 8
    m = wid // 8
    w0 = pl.multiple_of(288 - 96 * m, 8)
    pltpu.sync_copy(t_hbm.at[c, pl.ds(w0, WIN_ROWS)], win)
    copies = []
    for t in range(ROWS_PER_WORKER):
        copies.append(
            pltpu.async_copy(
                win.at[pl.ds(8 * (ROWS_PER_WORKER - 1 - t), L_OUT)],
                out_hbm.at[c + 96 * m + 8 * t],
                sem,
            )
        )
    for cp in copies:
        cp.wait()


def kernel(L, rel_pos_embed):
    t = pl.pallas_call(
        _phase_body,
        out_shape=jax.ShapeDtypeStruct((8, T_ROWS, D), jnp.float32),
    )(rel_pos_embed)
    mesh = plsc.VectorSubcoreMesh(core_axis_name="c", subcore_axis_name="s")
    run = pl.kernel(
        _pairwise_body,
        out_type=jax.ShapeDtypeStruct((L_OUT, L_OUT, D), jnp.float32),
        mesh=mesh,
        scratch_types=[
            pltpu.VMEM((WIN_ROWS, D), jnp.float32),
            pltpu.SemaphoreType.DMA,
        ],
    )
    return run(t)
